# baseline (speedup 1.0000x reference)
import functools

import jax
import jax.numpy as jnp
from jax import lax
from jax.experimental import pallas as pl
from jax.experimental.pallas import tpu as pltpu
from jax.experimental.pallas import tpu_sc as plsc

D = 64
SCALE = 8.0

_NC, _NS, _L = 2, 16, 16
_NW = _NC * _NS

CHUNK = 128
NBUF = 2


@functools.lru_cache(maxsize=None)
def _make_kernel(n_chunks, vrows):
    cpw = n_chunks // _NW
    assert cpw % NBUF == 0
    mesh = plsc.VectorSubcoreMesh(
        core_axis_name="c", subcore_axis_name="s",
        num_cores=_NC, num_subcores=_NS,
    )

    @functools.partial(
        pl.kernel,
        out_type=jax.ShapeDtypeStruct((n_chunks * D * CHUNK,), jnp.float32),
        mesh=mesh,
        scratch_types=(
            [pltpu.VMEM((CHUNK,), jnp.int32)] * NBUF +
            [pltpu.VMEM((CHUNK, D), jnp.float32)] * NBUF +
            [pltpu.VMEM((D * CHUNK,), jnp.float32)] * NBUF +
            [
                pltpu.SemaphoreType.DMA((NBUF,)),
                pltpu.SemaphoreType.DMA((NBUF,)),
                pltpu.SemaphoreType.DMA((NBUF,)),
            ]
        ),
        compiler_params=pltpu.CompilerParams(
            use_tc_tiling_on_sc=False, needs_layout_passes=False),
    )
    def emb_kernel(idx_hbm, tab_hbm, out_hbm, *refs):
        raw_v = refs[0:NBUF]
        g_v = refs[NBUF:2 * NBUF]
        t_v = refs[2 * NBUF:3 * NBUF]
        isem, gsem, osem = refs[3 * NBUF:]
        wid = lax.axis_index("s") * _NC + lax.axis_index("c")
        c0 = wid * cpw

        def idx_dma(k, b):
            return pltpu.make_async_copy(
                idx_hbm.at[c0 + k], raw_v[b], isem.at[b])

        def gather_dma(b):
            return pltpu.make_async_copy(
                tab_hbm.at[raw_v[b]], g_v[b], gsem.at[b])

        def out_dmas(k, b):
            c = c0 + k
            obase = (lax.shift_right_logical(c, 5) * 2048 + (c & 31) * 8) * CHUNK
            return [
                pltpu.make_async_copy(
                    t_v[b].at[pl.ds(dr * 8 * CHUNK, 8 * CHUNK)],
                    out_hbm.at[pl.ds(obase + dr * 256 * CHUNK, 8 * CHUNK)],
                    osem.at[b])
                for dr in range(8)
            ]

        def out_drain(b):
            for dr in range(8):
                pltpu.make_async_copy(
                    t_v[b].at[pl.ds(dr * 8 * CHUNK, 8 * CHUNK)],
                    out_hbm.at[pl.ds(dr * 8 * CHUNK, 8 * CHUNK)],
                    osem.at[b]).wait()

        def transpose(b):
            def cbody(c, carry):
                for q in range(D // _L):
                    vals = g_v[b][c, pl.ds(q * _L, _L)]
                    dest = (lax.iota(jnp.int32, _L) + (q * _L)) * CHUNK + c
                    plsc.store_scatter(t_v[b], [dest], vals * SCALE)
                return carry

            lax.fori_loop(0, CHUNK, cbody, 0, unroll=4)

        for b in range(NBUF):
            idx_dma(b, b).start()
        idx_dma(0, 0).wait()
        gather_dma(0).start()

        def outer(i, carry):
            for b in range(NBUF):
                k = i * NBUF + b
                bn = (b + 1) % NBUF
                gather_dma(b).wait()

                @pl.when(k + 1 < cpw)
                def _():
                    idx_dma(k + 1, bn).wait()
                    gather_dma(bn).start()

                @pl.when(k + NBUF < cpw)
                def _():
                    idx_dma(k + NBUF, b).start()

                @pl.when(k >= NBUF)
                def _():
                    out_drain(b)

                transpose(b)
                for d2 in out_dmas(k, b):
                    d2.start()
            return carry

        lax.fori_loop(0, cpw // NBUF, outer, 0)

        for b in range(NBUF):
            out_drain(b)

    return emb_kernel


def kernel(tokens, embed_table):
    s0, s1 = tokens.shape
    b = s0 * s1
    n_chunks = b // CHUNK
    idx = (tokens.T.reshape(n_chunks, CHUNK) * 2).astype(jnp.int32)
    tabp = jnp.pad(embed_table, ((0, 0), (0, 128 - D))).reshape(-1, D)
    out = _make_kernel(n_chunks, tabp.shape[0])(idx, tabp)
    return (out.reshape(s1, 8, s0 // CHUNK, D // 8, CHUNK)
               .transpose(2, 4, 0, 1, 3)
               .reshape(s0, s1, D))


# speedup vs baseline: 1.3259x; 1.3259x over previous
import functools

import jax
import jax.numpy as jnp
from jax import lax
from jax.experimental import pallas as pl
from jax.experimental.pallas import tpu as pltpu
from jax.experimental.pallas import tpu_sc as plsc

D = 64
SCALE = 8.0

_NC, _NS, _L = 2, 16, 16
_NW = _NC * _NS

IDXW = 128
CHUNK = 256
IDX_R = CHUNK // IDXW
NBUF = 3
ROW_UNROLL = 8


@functools.lru_cache(maxsize=None)
def _make_kernel(B, V):
    assert B % (_NW * CHUNK * NBUF) == 0 or B % (_NW * CHUNK) == 0
    b_per_w = B // _NW
    n_chunks = b_per_w // CHUNK
    assert n_chunks % NBUF == 1
    mesh = plsc.VectorSubcoreMesh(
        core_axis_name="c", subcore_axis_name="s",
        num_cores=_NC, num_subcores=_NS,
    )

    @functools.partial(
        pl.kernel,
        out_type=jax.ShapeDtypeStruct((B, D), jnp.float32),
        mesh=mesh,
        scratch_types=[
            pltpu.VMEM((NBUF, IDX_R, IDXW), jnp.int32),
            pltpu.VMEM((NBUF, CHUNK, D), jnp.float32),
            pltpu.SemaphoreType.DMA((NBUF,)),
            pltpu.SemaphoreType.DMA((NBUF,)),
            pltpu.SemaphoreType.DMA((NBUF,)),
        ],
        compiler_params=pltpu.CompilerParams(use_tc_tiling_on_sc=False),
    )
    def emb_kernel(idx_hbm, table_hbm, out_hbm, idx_v, rows_v, gsem, isem, osem):
        wid = lax.axis_index("s") * _NC + lax.axis_index("c")
        row0 = wid * (b_per_w // IDXW)
        out0 = wid * b_per_w

        def idx_src(cc):
            return idx_hbm.at[pl.ds(row0 + cc * IDX_R, IDX_R)]

        def out_dst(cc):
            return out_hbm.at[pl.ds(out0 + cc * CHUNK, CHUNK)]

        def gather_descs(b):
            return [
                pltpu.make_async_copy(
                    table_hbm.at[idx_v.at[b, j]],
                    rows_v.at[b, pl.ds(j * IDXW, IDXW)],
                    gsem.at[b],
                )
                for j in range(IDX_R)
            ]

        def scale(b):
            def body(r, carry):
                base = r * ROW_UNROLL
                for u in range(ROW_UNROLL):
                    for q in range(D // _L):
                        sl = pl.ds(q * _L, _L)
                        rows_v[b, base + u, sl] = rows_v[b, base + u, sl] * SCALE
                return carry
            lax.fori_loop(0, CHUNK // ROW_UNROLL, body, 0)

        for b in range(NBUF):
            pltpu.async_copy(idx_src(b), idx_v.at[b], isem.at[b])
        pltpu.make_async_copy(idx_src(0), idx_v.at[0], isem.at[0]).wait()
        for d in gather_descs(0):
            d.start()

        def tri_body(i, carry):
            for k in range(NBUF):
                cc = i * NBUF + k
                b, bn = k, (k + 1) % NBUF
                for d2 in gather_descs(b):
                    d2.wait()
                pltpu.make_async_copy(
                    idx_src(cc + 1), idx_v.at[bn], isem.at[bn]
                ).wait()
                @pl.when(cc >= NBUF - 1)
                def _():
                    pltpu.make_async_copy(
                        rows_v.at[bn], out_dst(cc), osem.at[bn]
                    ).wait()
                for d2 in gather_descs(bn):
                    d2.start()
                scale(b)
                pltpu.async_copy(rows_v.at[b], out_dst(cc), osem.at[b])
                @pl.when(cc + NBUF < n_chunks)
                def _():
                    pltpu.async_copy(idx_src(cc + NBUF), idx_v.at[b], isem.at[b])
            return carry

        lax.fori_loop(0, (n_chunks - 1) // NBUF, tri_body, 0)

        last = n_chunks - 1
        bl = last % NBUF
        for d in gather_descs(bl):
            d.wait()
        scale(bl)
        pltpu.async_copy(rows_v.at[bl], out_dst(last), osem.at[bl])
        for b in range(NBUF):
            pltpu.make_async_copy(
                rows_v.at[b], out_dst(last), osem.at[b]
            ).wait()

    return emb_kernel


def kernel(tokens, embed_table):
    s0, s1 = tokens.shape
    b = s0 * s1
    idx = tokens.reshape(b // IDXW, IDXW).astype(jnp.int32)
    out = _make_kernel(b, embed_table.shape[0])(idx, embed_table)
    return out.reshape(s0, s1, D)
